# baseline (speedup 1.0000x reference)
import functools

import jax
import jax.numpy as jnp
from jax import lax
from jax.experimental import pallas as pl
from jax.experimental.pallas import tpu as pltpu
from jax.experimental.pallas import tpu_sc as plsc

_NUM_CORES = 2
_NUM_SUBCORES = 16


def kernel(inputs):
    B, S, D = inputs.shape
    nw = _NUM_CORES * _NUM_SUBCORES
    per_row = nw // B
    chunk = D // per_row

    mesh = plsc.VectorSubcoreMesh(
        core_axis_name="c", subcore_axis_name="s")

    @functools.partial(
        pl.kernel,
        mesh=mesh,
        out_type=jax.ShapeDtypeStruct((B, D), inputs.dtype),
        scratch_types=[pltpu.VMEM((chunk,), inputs.dtype)],
    )
    def extract(x_hbm, o_hbm, buf):
        wid = lax.axis_index("s") * _NUM_CORES + lax.axis_index("c")
        b = wid // per_row
        off = (wid % per_row) * chunk
        pltpu.sync_copy(x_hbm.at[b, 0, pl.ds(off, chunk)], buf)
        pltpu.sync_copy(buf, o_hbm.at[b, pl.ds(off, chunk)])

    return extract(inputs)


# speedup vs baseline: 1.0183x; 1.0183x over previous
import functools

import jax
import jax.numpy as jnp
from jax import lax
from jax.experimental import pallas as pl
from jax.experimental.pallas import tpu as pltpu
from jax.experimental.pallas import tpu_sc as plsc

_NUM_CORES = 2


def kernel(inputs):
    B, S, D = inputs.shape
    rows_per_core = B // _NUM_CORES

    mesh = plsc.ScalarSubcoreMesh(axis_name="c", num_cores=_NUM_CORES)

    @functools.partial(
        pl.kernel,
        mesh=mesh,
        out_type=jax.ShapeDtypeStruct((B, D), inputs.dtype),
    )
    def extract(x_hbm, o_hbm):
        cid = lax.axis_index("c")
        for i in range(rows_per_core):
            b = cid * rows_per_core + i
            pltpu.sync_copy(x_hbm.at[b, 0], o_hbm.at[b])

    return extract(inputs)


# speedup vs baseline: 9.3881x; 9.2196x over previous
import jax
import jax.numpy as jnp
from jax.experimental import pallas as pl
from jax.experimental.pallas import tpu as pltpu


def _extract_body(x_hbm_ref, o_hbm_ref, sem):
    copy = pltpu.make_async_copy(x_hbm_ref.at[:, 0, :], o_hbm_ref, sem)
    copy.start()
    copy.wait()


def kernel(inputs):
    B, S, D = inputs.shape
    return pl.pallas_call(
        _extract_body,
        in_specs=[pl.BlockSpec(memory_space=pl.ANY)],
        out_specs=pl.BlockSpec(memory_space=pl.ANY),
        out_shape=jax.ShapeDtypeStruct((B, D), inputs.dtype),
        scratch_shapes=[pltpu.SemaphoreType.DMA],
    )(inputs)


# speedup vs baseline: 13.7827x; 1.4681x over previous
import jax
import jax.numpy as jnp
from jax.experimental import pallas as pl
from jax.experimental.pallas import tpu as pltpu


def _extract_body(x_hbm_ref, o_ref, sem):
    B = o_ref.shape[0]
    copies = [
        pltpu.make_async_copy(x_hbm_ref.at[b, 0, :], o_ref.at[b], sem)
        for b in range(B)
    ]
    for c in copies:
        c.start()
    for c in copies:
        c.wait()


def kernel(inputs):
    B, S, D = inputs.shape
    return pl.pallas_call(
        _extract_body,
        in_specs=[pl.BlockSpec(memory_space=pl.ANY)],
        out_specs=pl.BlockSpec((B, D), lambda: (0, 0)),
        out_shape=jax.ShapeDtypeStruct((B, D), inputs.dtype),
        scratch_shapes=[pltpu.SemaphoreType.DMA],
    )(inputs)
